# baseline (speedup 1.0000x reference)
import functools
import jax
import jax.numpy as jnp
from jax import lax
from jax.experimental import pallas as pl
from jax.experimental.pallas import tpu as pltpu
from jax.experimental.pallas import tpu_sc as plsc

N_NODES = 10000
N_EDGES = 320000
IN_DIM = 128
HID_DIM = 128
Z_DIM = 64

NC = 2
NS = 16
LANES = 16
NW = NC * NS

E_BLK = 80
N_BLK = 128
E_PER_W = N_BLK * E_BLK
E_PAD = NW * E_PER_W
AGG_ROWS = 10112
ROWS_PER_TILE = AGG_ROWS // NS

NRB = 4
CHUNK = 16
N_CHUNK = N_BLK // CHUNK


def _mm_body(x_ref, w_ref, b_ref, o_ref):
    acc = lax.dot_general(x_ref[...], w_ref[...],
                          (((1,), (1,)), ((), ())),
                          preferred_element_type=jnp.float32)
    o_ref[...] = acc + b_ref[...][None, :]


def _shared_linear(x, w, b):
    blk = 1000
    grid = N_NODES // blk
    return pl.pallas_call(
        _mm_body,
        grid=(grid,),
        in_specs=[
            pl.BlockSpec((blk, IN_DIM), lambda i: (i, 0)),
            pl.BlockSpec((HID_DIM, IN_DIM), lambda i: (0, 0)),
            pl.BlockSpec((HID_DIM,), lambda i: (0,)),
        ],
        out_specs=pl.BlockSpec((blk, HID_DIM), lambda i: (i, 0)),
        out_shape=jax.ShapeDtypeStruct((N_NODES, HID_DIM), jnp.float32),
    )(x, w, b)


def _spmm_body(h_hbm, src_hbm, dst_hbm, adj_hbm, out_hbm,
               src_c, dst_c, adj_c,
               b0_v, b1_v, b2_v, b3_v,
               gsem0, gsem1, gsem2, gsem3,
               ssem0, ssem1, ssem2, ssem3, agg_sh):
    cid = lax.axis_index("c")
    sid = lax.axis_index("s")
    wid = sid * NC + cid

    def zrow(i, _):
        for j in range(HID_DIM // LANES):
            b0_v[i, pl.ds(j * LANES, LANES)] = jnp.zeros((LANES,), jnp.float32)
        return 0
    lax.fori_loop(0, E_BLK, zrow, 0)
    rbase = sid * ROWS_PER_TILE
    for k in range(ROWS_PER_TILE // E_BLK):
        pltpu.sync_copy(b0_v, agg_sh.at[pl.ds(rbase + k * E_BLK, E_BLK)])
    rem = ROWS_PER_TILE % E_BLK
    if rem:
        pltpu.sync_copy(b0_v.at[pl.ds(0, rem)],
                        agg_sh.at[pl.ds(rbase + (ROWS_PER_TILE // E_BLK) * E_BLK, rem)])
    plsc.subcore_barrier()

    bufs = (b0_v, b1_v, b2_v, b3_v)
    gsems = (gsem0, gsem1, gsem2, gsem3)
    ssems = (ssem0, ssem1, ssem2, ssem3)

    def start_gather(m, q):
        pltpu.async_copy(h_hbm.at[src_c.at[m]], bufs[q], gsems[q])

    def wait_gather(q):
        pltpu.make_async_copy(h_hbm.at[src_c.at[0]], bufs[q], gsems[q]).wait()

    def start_scatter(m, q):
        pltpu.async_copy(bufs[q], agg_sh.at[dst_c.at[m]], ssems[q], add=True)

    def wait_scatter(q):
        pltpu.make_async_copy(bufs[q], agg_sh.at[dst_c.at[0]], ssems[q]).wait()

    def scale(m, q):
        buf = bufs[q]

        def group(g, _):
            av = adj_c[m, pl.ds(g * LANES, LANES)]
            for i in range(LANES):
                e = g * LANES + i
                sc = jnp.broadcast_to(av[i], (LANES,))
                for j in range(HID_DIM // LANES):
                    buf[e, pl.ds(j * LANES, LANES)] = buf[e, pl.ds(j * LANES, LANES)] * sc
            return 0
        lax.fori_loop(0, E_BLK // LANES, group, 0)

    def chunk(c, _):
        pltpu.sync_copy(src_hbm.at[wid].at[pl.ds(c * CHUNK, CHUNK)], src_c)
        pltpu.sync_copy(dst_hbm.at[wid].at[pl.ds(c * CHUNK, CHUNK)], dst_c)
        pltpu.sync_copy(adj_hbm.at[wid].at[pl.ds(c * CHUNK, CHUNK)], adj_c)

        def quad(p, _):
            m0 = NRB * p
            for q in range(NRB):
                start_gather(m0 + q, q)
            for q in range(NRB):
                wait_gather(q)
            for q in range(NRB):
                scale(m0 + q, q)
            for q in range(NRB):
                start_scatter(m0 + q, q)
            for q in range(NRB):
                wait_scatter(q)
            return 0

        lax.fori_loop(0, CHUNK // NRB, quad, 0)
        return 0

    lax.fori_loop(0, N_CHUNK, chunk, 0)
    plsc.subcore_barrier()

    pltpu.sync_copy(agg_sh.at[pl.ds(rbase, ROWS_PER_TILE)],
                    out_hbm.at[cid].at[pl.ds(rbase, ROWS_PER_TILE)])


def _spmm(h, src, dst, adj):
    mesh = plsc.VectorSubcoreMesh(core_axis_name="c", subcore_axis_name="s")
    k = functools.partial(
        pl.kernel,
        out_type=jax.ShapeDtypeStruct((NC, AGG_ROWS, HID_DIM), jnp.float32),
        mesh=mesh,
        scratch_types=[
            pltpu.VMEM((CHUNK, E_BLK), jnp.int32),
            pltpu.VMEM((CHUNK, E_BLK), jnp.int32),
            pltpu.VMEM((CHUNK, E_BLK), jnp.float32),
            pltpu.VMEM((E_BLK, HID_DIM), jnp.float32),
            pltpu.VMEM((E_BLK, HID_DIM), jnp.float32),
            pltpu.VMEM((E_BLK, HID_DIM), jnp.float32),
            pltpu.VMEM((E_BLK, HID_DIM), jnp.float32),
            pltpu.SemaphoreType.DMA,
            pltpu.SemaphoreType.DMA,
            pltpu.SemaphoreType.DMA,
            pltpu.SemaphoreType.DMA,
            pltpu.SemaphoreType.DMA,
            pltpu.SemaphoreType.DMA,
            pltpu.SemaphoreType.DMA,
            pltpu.SemaphoreType.DMA,
            pltpu.VMEM_SHARED((AGG_ROWS, HID_DIM), jnp.float32),
        ],
    )(_spmm_body)
    pad = E_PAD - N_EDGES
    src3 = jnp.concatenate([src, jnp.zeros((pad,), jnp.int32)]).reshape(NW, N_BLK, E_BLK)
    dst3 = jnp.concatenate([dst, jnp.zeros((pad,), jnp.int32)]).reshape(NW, N_BLK, E_BLK)
    adj3 = jnp.concatenate([adj, jnp.zeros((pad,), jnp.float32)]).reshape(NW, N_BLK, E_BLK)
    return k(h, src3, dst3, adj3)


def _heads_body(p0_ref, p1_ref, wm1_ref, bm1_ref, wm2_ref, bm2_ref,
                ws1_ref, bs1_ref, ws2_ref, bs2_ref, mean_ref, std_ref):
    hidden = jnp.maximum(p0_ref[...] + p1_ref[...], 0.0)
    dn = (((1,), (1,)), ((), ()))
    a = jnp.maximum(
        lax.dot_general(hidden, wm1_ref[...], dn, preferred_element_type=jnp.float32)
        + bm1_ref[...][None, :], 0.0)
    mean_ref[...] = (lax.dot_general(a, wm2_ref[...], dn, preferred_element_type=jnp.float32)
                     + bm2_ref[...][None, :])
    s = jnp.maximum(
        lax.dot_general(hidden, ws1_ref[...], dn, preferred_element_type=jnp.float32)
        + bs1_ref[...][None, :], 0.0)
    pre = (lax.dot_general(s, ws2_ref[...], dn, preferred_element_type=jnp.float32)
           + bs2_ref[...][None, :])
    std_ref[...] = jnp.maximum(pre, 0.0) + jnp.log1p(jnp.exp(-jnp.abs(pre)))


def _heads(partials, wm1, bm1, wm2, bm2, ws1, bs1, ws2, bs2):
    blk = 1264
    grid = AGG_ROWS // blk
    wspec = lambda shape: pl.BlockSpec(shape, lambda i: tuple(0 for _ in shape))
    mean, std = pl.pallas_call(
        _heads_body,
        grid=(grid,),
        in_specs=[
            pl.BlockSpec((blk, HID_DIM), lambda i: (i, 0)),
            pl.BlockSpec((blk, HID_DIM), lambda i: (i, 0)),
            wspec((Z_DIM, HID_DIM)), wspec((Z_DIM,)),
            wspec((Z_DIM, Z_DIM)), wspec((Z_DIM,)),
            wspec((Z_DIM, HID_DIM)), wspec((Z_DIM,)),
            wspec((Z_DIM, Z_DIM)), wspec((Z_DIM,)),
        ],
        out_specs=[
            pl.BlockSpec((blk, Z_DIM), lambda i: (i, 0)),
            pl.BlockSpec((blk, Z_DIM), lambda i: (i, 0)),
        ],
        out_shape=[
            jax.ShapeDtypeStruct((AGG_ROWS, Z_DIM), jnp.float32),
            jax.ShapeDtypeStruct((AGG_ROWS, Z_DIM), jnp.float32),
        ],
    )(partials[0], partials[1],
      wm1, bm1, wm2, bm2, ws1, bs1, ws2, bs2)
    return mean, std


def kernel(x, edge_index, adj_values, W_shared, b_shared,
           W_m1, b_m1, W_m2, b_m2, W_s1, b_s1, W_s2, b_s2):
    ei = edge_index.astype(jnp.int32)
    dst = ei[0]
    src = ei[1]
    h = _shared_linear(x, W_shared, b_shared)
    partials = _spmm(h, src, dst, adj_values)
    mean, std = _heads(partials, W_m1, b_m1, W_m2, b_m2, W_s1, b_s1, W_s2, b_s2)
    mean = mean[:N_NODES]
    std = std[:N_NODES]
    return (mean, mean, std)


# speedup vs baseline: 1.0200x; 1.0200x over previous
import functools
import jax
import jax.numpy as jnp
from jax import lax
from jax.experimental import pallas as pl
from jax.experimental.pallas import tpu as pltpu
from jax.experimental.pallas import tpu_sc as plsc

N_NODES = 10000
N_EDGES = 320000
IN_DIM = 128
HID_DIM = 128
Z_DIM = 64
HALF = 64

NC = 2
NS = 16
LANES = 16
NW = NC * NS

E_BLK = 80
N_BLK = 256
E_PER_W = N_BLK * E_BLK
E_PAD = NS * E_PER_W
PROWS = N_NODES // 2
AGG_PROWS = 5120
AGG_PER_TILE = AGG_PROWS // NS
HSTAGE = PROWS // NS

CHUNK = 16
N_CHUNK = N_BLK // CHUNK


def _mm_body(xe_ref, xo_ref, w_ref, b_ref, o_ref):
    dn = (((1,), (1,)), ((), ()))
    bias = b_ref[...][None, :]
    ev = lax.dot_general(xe_ref[...], w_ref[...], dn,
                         preferred_element_type=jnp.float32) + bias
    od = lax.dot_general(xo_ref[...], w_ref[...], dn,
                         preferred_element_type=jnp.float32) + bias
    o_ref[0] = jnp.concatenate([ev[:, :HALF], od[:, :HALF]], axis=1)
    o_ref[1] = jnp.concatenate([ev[:, HALF:], od[:, HALF:]], axis=1)


def _shared_linear(x, w, b):
    blk = 1000
    grid = PROWS // blk
    xr = x.reshape(PROWS, 2, IN_DIM)
    xe = xr[:, 0, :]
    xo = xr[:, 1, :]
    return pl.pallas_call(
        _mm_body,
        grid=(grid,),
        in_specs=[
            pl.BlockSpec((blk, IN_DIM), lambda i: (i, 0)),
            pl.BlockSpec((blk, IN_DIM), lambda i: (i, 0)),
            pl.BlockSpec((HID_DIM, IN_DIM), lambda i: (0, 0)),
            pl.BlockSpec((HID_DIM,), lambda i: (0,)),
        ],
        out_specs=pl.BlockSpec((2, blk, HID_DIM), lambda i: (0, i, 0)),
        out_shape=jax.ShapeDtypeStruct((2, PROWS, HID_DIM), jnp.float32),
    )(xe, xo, w, b)


def _spmm_body(h_hbm, src_hbm, dst_hbm, off_hbm, adj_hbm, out_hbm,
               src_c, dst_c, off_c, adj_c,
               g0_v, g1_v, s0_v, s1_v,
               gsem0, gsem1, ssem0, ssem1,
               h_sp, agg_sp):
    cid = lax.axis_index("c")
    sid = lax.axis_index("s")

    def zrow(i, _):
        for j in range(HID_DIM // LANES):
            g0_v[i, pl.ds(j * LANES, LANES)] = jnp.zeros((LANES,), jnp.float32)
        return 0
    lax.fori_loop(0, E_BLK, zrow, 0)
    abase = sid * AGG_PER_TILE
    for k in range(AGG_PER_TILE // E_BLK):
        pltpu.sync_copy(g0_v, agg_sp.at[pl.ds(abase + k * E_BLK, E_BLK)])

    pltpu.sync_copy(h_hbm.at[cid].at[pl.ds(sid * HSTAGE, HSTAGE)],
                    h_sp.at[pl.ds(sid * HSTAGE, HSTAGE)])

    @pl.when(sid == NS - 1)
    def _():
        pltpu.sync_copy(h_hbm.at[cid].at[pl.ds(NS * HSTAGE, PROWS - NS * HSTAGE)],
                        h_sp.at[pl.ds(NS * HSTAGE, PROWS - NS * HSTAGE)])
    plsc.subcore_barrier()

    gbufs = (g0_v, g1_v)
    gsems = (gsem0, gsem1)
    sbufs = (s0_v, s1_v)
    ssems = (ssem0, ssem1)
    wid = cid * NS + sid

    def start_gather(m, q):
        pltpu.async_copy(h_sp.at[src_c.at[m]], gbufs[q], gsems[q])

    def wait_gather(q):
        pltpu.make_async_copy(h_sp.at[src_c.at[0]], gbufs[q], gsems[q]).wait()

    def start_scatter(m, q):
        pltpu.async_copy(sbufs[q], agg_sp.at[dst_c.at[m]], ssems[q], add=True)

    def wait_scatter(q):
        pltpu.make_async_copy(sbufs[q], agg_sp.at[dst_c.at[0]], ssems[q]).wait()

    def scale(m, q):
        gbuf = gbufs[q]
        sbuf = sbufs[q]
        zero = jnp.zeros((LANES,), jnp.float32)

        def group(g, _):
            base = g * LANES
            av = adj_c[m, pl.ds(base, LANES)]
            ov = off_c[m, pl.ds(base, LANES)]
            sov = jnp.bitwise_and(ov, 255)
            dov = jnp.right_shift(ov, 8)
            for i in range(LANES):
                e = base + i
                sc = jnp.broadcast_to(av[i], (LANES,))
                so = sov[i]
                do = dov[i]
                dz = HALF - do
                for j in range(HALF // LANES):
                    sbuf[e, pl.ds(do + j * LANES, LANES)] = (
                        gbuf[e, pl.ds(so + j * LANES, LANES)] * sc)
                    sbuf[e, pl.ds(dz + j * LANES, LANES)] = zero
            return 0
        lax.fori_loop(0, E_BLK // LANES, group, 0)

    def chunk(c, _):
        pltpu.sync_copy(src_hbm.at[wid].at[pl.ds(c * CHUNK, CHUNK)], src_c)
        pltpu.sync_copy(dst_hbm.at[wid].at[pl.ds(c * CHUNK, CHUNK)], dst_c)
        pltpu.sync_copy(off_hbm.at[wid].at[pl.ds(c * CHUNK, CHUNK)], off_c)
        pltpu.sync_copy(adj_hbm.at[wid].at[pl.ds(c * CHUNK, CHUNK)], adj_c)

        start_gather(0, 0)
        start_gather(1, 1)

        def pair(p, _):
            m0 = 2 * p
            for q in range(2):
                m = m0 + q
                wait_gather(q)

                @pl.when(p > 0)
                def _():
                    wait_scatter(q)
                scale(m, q)
                start_scatter(m, q)

                @pl.when(p < CHUNK // 2 - 1)
                def _():
                    start_gather(m + 2, q)
            return 0

        lax.fori_loop(0, CHUNK // 2, pair, 0)
        wait_scatter(0)
        wait_scatter(1)
        return 0

    lax.fori_loop(0, N_CHUNK, chunk, 0)
    plsc.subcore_barrier()

    pltpu.sync_copy(agg_sp.at[pl.ds(abase, AGG_PER_TILE)],
                    out_hbm.at[cid].at[pl.ds(abase, AGG_PER_TILE)])


def _spmm(hp, src, dst, adj):
    mesh = plsc.VectorSubcoreMesh(core_axis_name="c", subcore_axis_name="s")
    k = functools.partial(
        pl.kernel,
        out_type=jax.ShapeDtypeStruct((NC, AGG_PROWS, HID_DIM), jnp.float32),
        mesh=mesh,
        scratch_types=[
            pltpu.VMEM((CHUNK, E_BLK), jnp.int32),
            pltpu.VMEM((CHUNK, E_BLK), jnp.int32),
            pltpu.VMEM((CHUNK, E_BLK), jnp.int32),
            pltpu.VMEM((CHUNK, E_BLK), jnp.float32),
            pltpu.VMEM((E_BLK, HID_DIM), jnp.float32),
            pltpu.VMEM((E_BLK, HID_DIM), jnp.float32),
            pltpu.VMEM((E_BLK, HID_DIM), jnp.float32),
            pltpu.VMEM((E_BLK, HID_DIM), jnp.float32),
            pltpu.SemaphoreType.DMA,
            pltpu.SemaphoreType.DMA,
            pltpu.SemaphoreType.DMA,
            pltpu.SemaphoreType.DMA,
            pltpu.VMEM_SHARED((PROWS, HID_DIM), jnp.float32),
            pltpu.VMEM_SHARED((AGG_PROWS, HID_DIM), jnp.float32),
        ],
    )(_spmm_body)
    pad = E_PAD - N_EDGES
    srcp = jnp.concatenate([src, jnp.zeros((pad,), jnp.int32)])
    dstp = jnp.concatenate([dst, jnp.zeros((pad,), jnp.int32)])
    adjp = jnp.concatenate([adj, jnp.zeros((pad,), jnp.float32)])
    srow = (srcp // 2).reshape(NS, N_BLK, E_BLK)
    drow = (dstp // 2).reshape(NS, N_BLK, E_BLK)
    offp = ((srcp & 1) * HALF + (((dstp & 1) * HALF) << 8)).reshape(NS, N_BLK, E_BLK)
    adjp = adjp.reshape(NS, N_BLK, E_BLK)
    dup = lambda a: jnp.broadcast_to(a[None], (NC, NS, N_BLK, E_BLK)).reshape(NW, N_BLK, E_BLK)
    return k(hp, dup(srow), dup(drow), dup(offp), dup(adjp))


def _heads_body(p0_ref, p1_ref, wm1_ref, bm1_ref, wm2_ref, bm2_ref,
                ws1_ref, bs1_ref, ws2_ref, bs2_ref, mean_ref, std_ref):
    p0 = p0_ref[...]
    p1 = p1_ref[...]
    ev = jnp.concatenate([p0[:, :HALF], p1[:, :HALF]], axis=1)
    od = jnp.concatenate([p0[:, HALF:], p1[:, HALF:]], axis=1)
    hidden = jnp.stack([ev, od], axis=1).reshape(2 * p0.shape[0], HID_DIM)
    hidden = jnp.maximum(hidden, 0.0)
    dn = (((1,), (1,)), ((), ()))
    a = jnp.maximum(
        lax.dot_general(hidden, wm1_ref[...], dn, preferred_element_type=jnp.float32)
        + bm1_ref[...][None, :], 0.0)
    mean_ref[...] = (lax.dot_general(a, wm2_ref[...], dn, preferred_element_type=jnp.float32)
                     + bm2_ref[...][None, :])
    s = jnp.maximum(
        lax.dot_general(hidden, ws1_ref[...], dn, preferred_element_type=jnp.float32)
        + bs1_ref[...][None, :], 0.0)
    pre = (lax.dot_general(s, ws2_ref[...], dn, preferred_element_type=jnp.float32)
           + bs2_ref[...][None, :])
    std_ref[...] = jnp.maximum(pre, 0.0) + jnp.log1p(jnp.exp(-jnp.abs(pre)))


def _heads(partials, wm1, bm1, wm2, bm2, ws1, bs1, ws2, bs2):
    blk = 640
    grid = AGG_PROWS // blk
    wspec = lambda shape: pl.BlockSpec(shape, lambda i: tuple(0 for _ in shape))
    mean, std = pl.pallas_call(
        _heads_body,
        grid=(grid,),
        in_specs=[
            pl.BlockSpec((blk, HID_DIM), lambda i: (i, 0)),
            pl.BlockSpec((blk, HID_DIM), lambda i: (i, 0)),
            wspec((Z_DIM, HID_DIM)), wspec((Z_DIM,)),
            wspec((Z_DIM, Z_DIM)), wspec((Z_DIM,)),
            wspec((Z_DIM, HID_DIM)), wspec((Z_DIM,)),
            wspec((Z_DIM, Z_DIM)), wspec((Z_DIM,)),
        ],
        out_specs=[
            pl.BlockSpec((2 * blk, Z_DIM), lambda i: (i, 0)),
            pl.BlockSpec((2 * blk, Z_DIM), lambda i: (i, 0)),
        ],
        out_shape=[
            jax.ShapeDtypeStruct((2 * AGG_PROWS, Z_DIM), jnp.float32),
            jax.ShapeDtypeStruct((2 * AGG_PROWS, Z_DIM), jnp.float32),
        ],
    )(partials[0], partials[1],
      wm1, bm1, wm2, bm2, ws1, bs1, ws2, bs2)
    return mean, std


def kernel(x, edge_index, adj_values, W_shared, b_shared,
           W_m1, b_m1, W_m2, b_m2, W_s1, b_s1, W_s2, b_s2):
    ei = edge_index.astype(jnp.int32)
    dst = ei[0]
    src = ei[1]
    hp = _shared_linear(x, W_shared, b_shared)
    partials = _spmm(hp, src, dst, adj_values)
    mean, std = _heads(partials, W_m1, b_m1, W_m2, b_m2, W_s1, b_s1, W_s2, b_s2)
    mean = mean[:N_NODES]
    std = std[:N_NODES]
    return (mean, mean, std)


# speedup vs baseline: 1.3041x; 1.2785x over previous
import functools
import jax
import jax.numpy as jnp
from jax import lax
from jax.experimental import pallas as pl
from jax.experimental.pallas import tpu as pltpu
from jax.experimental.pallas import tpu_sc as plsc

N_NODES = 10000
N_EDGES = 320000
IN_DIM = 128
HID_DIM = 128
Z_DIM = 64

NC = 2
NS = 16
LANES = 16
NW = NC * NS

E_PER_W = N_EDGES // NW
E_BLK = 80
N_BLK = E_PER_W // E_BLK
AGG_ROWS = 10240
ROWS_PER_TILE = AGG_ROWS // NS
ZCHUNK = 80


def _mm_body(x_ref, w_ref, b_ref, o_ref):
    acc = lax.dot_general(x_ref[...], w_ref[...],
                          (((1,), (1,)), ((), ())),
                          preferred_element_type=jnp.float32)
    o_ref[...] = acc + b_ref[...][None, :]


def _shared_linear(x, w, b):
    blk = 1000
    grid = N_NODES // blk
    return pl.pallas_call(
        _mm_body,
        grid=(grid,),
        in_specs=[
            pl.BlockSpec((blk, IN_DIM), lambda i: (i, 0)),
            pl.BlockSpec((HID_DIM, IN_DIM), lambda i: (0, 0)),
            pl.BlockSpec((HID_DIM,), lambda i: (0,)),
        ],
        out_specs=pl.BlockSpec((blk, HID_DIM), lambda i: (i, 0)),
        out_shape=jax.ShapeDtypeStruct((N_NODES, HID_DIM), jnp.float32),
    )(x, w, b)


def _spmm_body(h_hbm, src_hbm, dst_hbm, adj_hbm, out_hbm,
               src_v, dst_v, adj_v, rows_v, zbuf_v, agg_sh, sem):
    cid = lax.axis_index("c")
    sid = lax.axis_index("s")
    wid = sid * NC + cid

    def zrow(i, _):
        for j in range(HID_DIM // LANES):
            zbuf_v[i, pl.ds(j * LANES, LANES)] = jnp.zeros((LANES,), jnp.float32)
        return 0
    lax.fori_loop(0, ZCHUNK, zrow, 0)
    for k in range(ROWS_PER_TILE // ZCHUNK):
        pltpu.sync_copy(zbuf_v, agg_sh.at[pl.ds(sid * ROWS_PER_TILE + k * ZCHUNK, ZCHUNK)])
    plsc.subcore_barrier()

    ebase = wid * E_PER_W

    def block(b, _):
        base = ebase + b * E_BLK
        pltpu.sync_copy(src_hbm.at[pl.ds(base, E_BLK)], src_v)
        pltpu.sync_copy(dst_hbm.at[pl.ds(base, E_BLK)], dst_v)
        pltpu.sync_copy(adj_hbm.at[pl.ds(base, E_BLK)], adj_v)
        pltpu.async_copy(h_hbm.at[src_v], rows_v, sem).wait()

        def group(g, _):
            av = adj_v[pl.ds(g * LANES, LANES)]
            for i in range(LANES):
                e = g * LANES + i
                scale = jnp.broadcast_to(av[i], (LANES,))
                for j in range(HID_DIM // LANES):
                    seg = rows_v[e, pl.ds(j * LANES, LANES)]
                    rows_v[e, pl.ds(j * LANES, LANES)] = seg * scale
            return 0
        lax.fori_loop(0, E_BLK // LANES, group, 0)

        pltpu.sync_copy(rows_v, agg_sh.at[dst_v], add=True)
        return 0

    lax.fori_loop(0, N_BLK, block, 0)
    plsc.subcore_barrier()

    rbase = sid * ROWS_PER_TILE
    pltpu.sync_copy(agg_sh.at[pl.ds(rbase, ROWS_PER_TILE)],
                    out_hbm.at[cid].at[pl.ds(rbase, ROWS_PER_TILE)])


def _spmm(h, src, dst, adj):
    mesh = plsc.VectorSubcoreMesh(core_axis_name="c", subcore_axis_name="s")
    k = functools.partial(
        pl.kernel,
        out_type=jax.ShapeDtypeStruct((NC, AGG_ROWS, HID_DIM), jnp.float32),
        mesh=mesh,
        scratch_types=[
            pltpu.VMEM((E_BLK,), jnp.int32),
            pltpu.VMEM((E_BLK,), jnp.int32),
            pltpu.VMEM((E_BLK,), jnp.float32),
            pltpu.VMEM((E_BLK, HID_DIM), jnp.float32),
            pltpu.VMEM((ZCHUNK, HID_DIM), jnp.float32),
            pltpu.VMEM_SHARED((AGG_ROWS, HID_DIM), jnp.float32),
            pltpu.SemaphoreType.DMA,
        ],
    )(_spmm_body)
    return k(h, src, dst, adj)


def _heads_body(p0_ref, p1_ref, wm1_ref, bm1_ref, wm2_ref, bm2_ref,
                ws1_ref, bs1_ref, ws2_ref, bs2_ref, mean_ref, std_ref):
    hidden = jnp.maximum(p0_ref[...] + p1_ref[...], 0.0)
    dn = (((1,), (1,)), ((), ()))
    a = jnp.maximum(
        lax.dot_general(hidden, wm1_ref[...], dn, preferred_element_type=jnp.float32)
        + bm1_ref[...][None, :], 0.0)
    mean_ref[...] = (lax.dot_general(a, wm2_ref[...], dn, preferred_element_type=jnp.float32)
                     + bm2_ref[...][None, :])
    s = jnp.maximum(
        lax.dot_general(hidden, ws1_ref[...], dn, preferred_element_type=jnp.float32)
        + bs1_ref[...][None, :], 0.0)
    pre = (lax.dot_general(s, ws2_ref[...], dn, preferred_element_type=jnp.float32)
           + bs2_ref[...][None, :])
    std_ref[...] = jnp.maximum(pre, 0.0) + jnp.log1p(jnp.exp(-jnp.abs(pre)))


def _heads(partials, wm1, bm1, wm2, bm2, ws1, bs1, ws2, bs2):
    blk = 1024
    grid = AGG_ROWS // blk
    wspec = lambda shape: pl.BlockSpec(shape, lambda i: tuple(0 for _ in shape))
    mean, std = pl.pallas_call(
        _heads_body,
        grid=(grid,),
        in_specs=[
            pl.BlockSpec((blk, HID_DIM), lambda i: (i, 0)),
            pl.BlockSpec((blk, HID_DIM), lambda i: (i, 0)),
            wspec((Z_DIM, HID_DIM)), wspec((Z_DIM,)),
            wspec((Z_DIM, Z_DIM)), wspec((Z_DIM,)),
            wspec((Z_DIM, HID_DIM)), wspec((Z_DIM,)),
            wspec((Z_DIM, Z_DIM)), wspec((Z_DIM,)),
        ],
        out_specs=[
            pl.BlockSpec((blk, Z_DIM), lambda i: (i, 0)),
            pl.BlockSpec((blk, Z_DIM), lambda i: (i, 0)),
        ],
        out_shape=[
            jax.ShapeDtypeStruct((AGG_ROWS, Z_DIM), jnp.float32),
            jax.ShapeDtypeStruct((AGG_ROWS, Z_DIM), jnp.float32),
        ],
    )(partials[0], partials[1],
      wm1, bm1, wm2, bm2, ws1, bs1, ws2, bs2)
    return mean, std


def kernel(x, edge_index, adj_values, W_shared, b_shared,
           W_m1, b_m1, W_m2, b_m2, W_s1, b_s1, W_s2, b_s2):
    ei = edge_index.astype(jnp.int32)
    dst = ei[0]
    src = ei[1]
    h = _shared_linear(x, W_shared, b_shared)
    partials = _spmm(h, src, dst, adj_values)
    mean, std = _heads(partials, W_m1, b_m1, W_m2, b_m2, W_s1, b_s1, W_s2, b_s2)
    mean = mean[:N_NODES]
    std = std[:N_NODES]
    return (mean, mean, std)


# speedup vs baseline: 1.7769x; 1.3626x over previous
import functools
import jax
import jax.numpy as jnp
from jax import lax
from jax.experimental import pallas as pl
from jax.experimental.pallas import tpu as pltpu
from jax.experimental.pallas import tpu_sc as plsc

N_NODES = 10000
N_EDGES = 320000
IN_DIM = 128
HID_DIM = 128
Z_DIM = 64

NC = 2
NS = 16
LANES = 16
NW = NC * NS

E_PER_W = N_EDGES // NW
E_BLK = 80
N_BLK = E_PER_W // E_BLK
AGG_ROWS = 10240
ROWS_PER_TILE = AGG_ROWS // NS
ZCHUNK = 80


def _mm_body(x_ref, w_ref, b_ref, o_ref):
    acc = lax.dot_general(x_ref[...], w_ref[...],
                          (((1,), (1,)), ((), ())),
                          preferred_element_type=jnp.float32)
    o_ref[...] = acc + b_ref[...][None, :]


def _shared_linear(x, w, b):
    blk = 1000
    grid = N_NODES // blk
    return pl.pallas_call(
        _mm_body,
        grid=(grid,),
        in_specs=[
            pl.BlockSpec((blk, IN_DIM), lambda i: (i, 0)),
            pl.BlockSpec((HID_DIM, IN_DIM), lambda i: (0, 0)),
            pl.BlockSpec((HID_DIM,), lambda i: (0,)),
        ],
        out_specs=pl.BlockSpec((blk, HID_DIM), lambda i: (i, 0)),
        out_shape=jax.ShapeDtypeStruct((N_NODES, HID_DIM), jnp.float32),
    )(x, w, b)


def _spmm_body(h_hbm, src_hbm, dst_hbm, adj_hbm, out_hbm,
               src0_v, dst0_v, adj0_v, rows0_v,
               src1_v, dst1_v, adj1_v, rows1_v,
               zbuf_v, agg_sh, gsem0, gsem1, ssem0, ssem1):
    cid = lax.axis_index("c")
    sid = lax.axis_index("s")
    wid = sid * NC + cid

    def zrow(i, _):
        for j in range(HID_DIM // LANES):
            zbuf_v[i, pl.ds(j * LANES, LANES)] = jnp.zeros((LANES,), jnp.float32)
        return 0
    lax.fori_loop(0, ZCHUNK, zrow, 0)
    for k in range(ROWS_PER_TILE // ZCHUNK):
        pltpu.sync_copy(zbuf_v, agg_sh.at[pl.ds(sid * ROWS_PER_TILE + k * ZCHUNK, ZCHUNK)])
    plsc.subcore_barrier()

    ebase = wid * E_PER_W
    srcs = (src0_v, src1_v)
    dsts = (dst0_v, dst1_v)
    adjs = (adj0_v, adj1_v)
    rows = (rows0_v, rows1_v)
    gsems = (gsem0, gsem1)
    ssems = (ssem0, ssem1)

    def stage(b, s, first):
        if not first:
            pltpu.make_async_copy(rows[s], agg_sh.at[dsts[s]], ssems[s]).wait()
        base = ebase + b * E_BLK
        pltpu.sync_copy(src_hbm.at[pl.ds(base, E_BLK)], srcs[s])
        pltpu.sync_copy(dst_hbm.at[pl.ds(base, E_BLK)], dsts[s])
        pltpu.sync_copy(adj_hbm.at[pl.ds(base, E_BLK)], adjs[s])
        pltpu.async_copy(h_hbm.at[srcs[s]], rows[s], gsems[s])

    def compute(s):
        pltpu.make_async_copy(h_hbm.at[srcs[s]], rows[s], gsems[s]).wait()
        rv = rows[s]
        av_ref = adjs[s]

        def group(g, _):
            av = av_ref[pl.ds(g * LANES, LANES)]
            for i in range(LANES):
                e = g * LANES + i
                scale = jnp.broadcast_to(av[i], (LANES,))
                for j in range(HID_DIM // LANES):
                    seg = rv[e, pl.ds(j * LANES, LANES)]
                    rv[e, pl.ds(j * LANES, LANES)] = seg * scale
            return 0
        lax.fori_loop(0, E_BLK // LANES, group, 0)
        pltpu.async_copy(rv, agg_sh.at[dsts[s]], ssems[s], add=True)

    stage(0, 0, True)
    stage(1, 1, True)

    def pair(p, _):
        b0 = 2 * p
        compute(0)
        stage(b0 + 2, 0, False)
        compute(1)

        @pl.when(p < (N_BLK - 1) // 2 - 1)
        def _():
            stage(b0 + 3, 1, False)
        return 0

    lax.fori_loop(0, (N_BLK - 1) // 2, pair, 0)
    compute(0)
    pltpu.make_async_copy(rows[0], agg_sh.at[dsts[0]], ssems[0]).wait()
    pltpu.make_async_copy(rows[1], agg_sh.at[dsts[1]], ssems[1]).wait()
    plsc.subcore_barrier()

    rbase = sid * ROWS_PER_TILE
    pltpu.sync_copy(agg_sh.at[pl.ds(rbase, ROWS_PER_TILE)],
                    out_hbm.at[cid].at[pl.ds(rbase, ROWS_PER_TILE)])


def _spmm(h, src, dst, adj):
    mesh = plsc.VectorSubcoreMesh(core_axis_name="c", subcore_axis_name="s")
    k = functools.partial(
        pl.kernel,
        out_type=jax.ShapeDtypeStruct((NC, AGG_ROWS, HID_DIM), jnp.float32),
        mesh=mesh,
        scratch_types=[
            pltpu.VMEM((E_BLK,), jnp.int32),
            pltpu.VMEM((E_BLK,), jnp.int32),
            pltpu.VMEM((E_BLK,), jnp.float32),
            pltpu.VMEM((E_BLK, HID_DIM), jnp.float32),
            pltpu.VMEM((E_BLK,), jnp.int32),
            pltpu.VMEM((E_BLK,), jnp.int32),
            pltpu.VMEM((E_BLK,), jnp.float32),
            pltpu.VMEM((E_BLK, HID_DIM), jnp.float32),
            pltpu.VMEM((ZCHUNK, HID_DIM), jnp.float32),
            pltpu.VMEM_SHARED((AGG_ROWS, HID_DIM), jnp.float32),
            pltpu.SemaphoreType.DMA,
            pltpu.SemaphoreType.DMA,
            pltpu.SemaphoreType.DMA,
            pltpu.SemaphoreType.DMA,
        ],
    )(_spmm_body)
    return k(h, src, dst, adj)


def _heads_body(p0_ref, p1_ref, wm1_ref, bm1_ref, wm2_ref, bm2_ref,
                ws1_ref, bs1_ref, ws2_ref, bs2_ref, mean_ref, std_ref):
    hidden = jnp.maximum(p0_ref[...] + p1_ref[...], 0.0)
    dn = (((1,), (1,)), ((), ()))
    a = jnp.maximum(
        lax.dot_general(hidden, wm1_ref[...], dn, preferred_element_type=jnp.float32)
        + bm1_ref[...][None, :], 0.0)
    mean_ref[...] = (lax.dot_general(a, wm2_ref[...], dn, preferred_element_type=jnp.float32)
                     + bm2_ref[...][None, :])
    s = jnp.maximum(
        lax.dot_general(hidden, ws1_ref[...], dn, preferred_element_type=jnp.float32)
        + bs1_ref[...][None, :], 0.0)
    pre = (lax.dot_general(s, ws2_ref[...], dn, preferred_element_type=jnp.float32)
           + bs2_ref[...][None, :])
    std_ref[...] = jnp.maximum(pre, 0.0) + jnp.log1p(jnp.exp(-jnp.abs(pre)))


def _heads(partials, wm1, bm1, wm2, bm2, ws1, bs1, ws2, bs2):
    blk = 1024
    grid = AGG_ROWS // blk
    wspec = lambda shape: pl.BlockSpec(shape, lambda i: tuple(0 for _ in shape))
    mean, std = pl.pallas_call(
        _heads_body,
        grid=(grid,),
        in_specs=[
            pl.BlockSpec((blk, HID_DIM), lambda i: (i, 0)),
            pl.BlockSpec((blk, HID_DIM), lambda i: (i, 0)),
            wspec((Z_DIM, HID_DIM)), wspec((Z_DIM,)),
            wspec((Z_DIM, Z_DIM)), wspec((Z_DIM,)),
            wspec((Z_DIM, HID_DIM)), wspec((Z_DIM,)),
            wspec((Z_DIM, Z_DIM)), wspec((Z_DIM,)),
        ],
        out_specs=[
            pl.BlockSpec((blk, Z_DIM), lambda i: (i, 0)),
            pl.BlockSpec((blk, Z_DIM), lambda i: (i, 0)),
        ],
        out_shape=[
            jax.ShapeDtypeStruct((AGG_ROWS, Z_DIM), jnp.float32),
            jax.ShapeDtypeStruct((AGG_ROWS, Z_DIM), jnp.float32),
        ],
    )(partials[0], partials[1],
      wm1, bm1, wm2, bm2, ws1, bs1, ws2, bs2)
    return mean, std


def kernel(x, edge_index, adj_values, W_shared, b_shared,
           W_m1, b_m1, W_m2, b_m2, W_s1, b_s1, W_s2, b_s2):
    ei = edge_index.astype(jnp.int32)
    dst = ei[0]
    src = ei[1]
    h = _shared_linear(x, W_shared, b_shared)
    partials = _spmm(h, src, dst, adj_values)
    mean, std = _heads(partials, W_m1, b_m1, W_m2, b_m2, W_s1, b_s1, W_s2, b_s2)
    mean = mean[:N_NODES]
    std = std[:N_NODES]
    return (mean, mean, std)


# speedup vs baseline: 1.7816x; 1.0027x over previous
import functools
import jax
import jax.numpy as jnp
from jax import lax
from jax.experimental import pallas as pl
from jax.experimental.pallas import tpu as pltpu
from jax.experimental.pallas import tpu_sc as plsc

N_NODES = 10000
N_EDGES = 320000
IN_DIM = 128
HID_DIM = 128
Z_DIM = 64

NC = 2
NS = 16
LANES = 16
NW = NC * NS

E_PER_W = N_EDGES // NW
E_BLK = 80
N_BLK = E_PER_W // E_BLK
AGG_ROWS = 10240
ROWS_PER_TILE = AGG_ROWS // NS
ZCHUNK = 80


def _mm_body(x_ref, w_ref, b_ref, o_ref):
    acc = lax.dot_general(x_ref[...], w_ref[...],
                          (((1,), (1,)), ((), ())),
                          preferred_element_type=jnp.float32)
    o_ref[...] = acc + b_ref[...][None, :]


def _shared_linear(x, w, b):
    blk = 1000
    grid = N_NODES // blk
    return pl.pallas_call(
        _mm_body,
        grid=(grid,),
        in_specs=[
            pl.BlockSpec((blk, IN_DIM), lambda i: (i, 0)),
            pl.BlockSpec((HID_DIM, IN_DIM), lambda i: (0, 0)),
            pl.BlockSpec((HID_DIM,), lambda i: (0,)),
        ],
        out_specs=pl.BlockSpec((blk, HID_DIM), lambda i: (i, 0)),
        out_shape=jax.ShapeDtypeStruct((N_NODES, HID_DIM), jnp.float32),
    )(x, w, b)


NSLOT = 3


def _spmm_body(h_hbm, src_hbm, dst_hbm, adj_hbm, out_hbm,
               src0_v, dst0_v, adj0_v, rows0_v,
               src1_v, dst1_v, adj1_v, rows1_v,
               src2_v, dst2_v, adj2_v, rows2_v,
               zbuf_v, agg_sh, gsem0, gsem1, gsem2, ssem0, ssem1, ssem2):
    cid = lax.axis_index("c")
    sid = lax.axis_index("s")
    wid = sid * NC + cid

    def zrow(i, _):
        for j in range(HID_DIM // LANES):
            zbuf_v[i, pl.ds(j * LANES, LANES)] = jnp.zeros((LANES,), jnp.float32)
        return 0
    lax.fori_loop(0, ZCHUNK, zrow, 0)
    for k in range(ROWS_PER_TILE // ZCHUNK):
        pltpu.sync_copy(zbuf_v, agg_sh.at[pl.ds(sid * ROWS_PER_TILE + k * ZCHUNK, ZCHUNK)])
    plsc.subcore_barrier()

    ebase = wid * E_PER_W
    srcs = (src0_v, src1_v, src2_v)
    dsts = (dst0_v, dst1_v, dst2_v)
    adjs = (adj0_v, adj1_v, adj2_v)
    rows = (rows0_v, rows1_v, rows2_v)
    gsems = (gsem0, gsem1, gsem2)
    ssems = (ssem0, ssem1, ssem2)

    def stage(b, s, first):
        if not first:
            pltpu.make_async_copy(rows[s], agg_sh.at[dsts[s]], ssems[s]).wait()
        base = ebase + b * E_BLK
        pltpu.sync_copy(src_hbm.at[pl.ds(base, E_BLK)], srcs[s])
        pltpu.sync_copy(dst_hbm.at[pl.ds(base, E_BLK)], dsts[s])
        pltpu.sync_copy(adj_hbm.at[pl.ds(base, E_BLK)], adjs[s])
        pltpu.async_copy(h_hbm.at[srcs[s]], rows[s], gsems[s])

    def compute(s):
        pltpu.make_async_copy(h_hbm.at[srcs[s]], rows[s], gsems[s]).wait()
        rv = rows[s]
        av_ref = adjs[s]

        def group(g, _):
            av = av_ref[pl.ds(g * LANES, LANES)]
            for i in range(LANES):
                e = g * LANES + i
                scale = jnp.broadcast_to(av[i], (LANES,))
                for j in range(HID_DIM // LANES):
                    seg = rv[e, pl.ds(j * LANES, LANES)]
                    rv[e, pl.ds(j * LANES, LANES)] = seg * scale
            return 0
        lax.fori_loop(0, E_BLK // LANES, group, 0)
        pltpu.async_copy(rv, agg_sh.at[dsts[s]], ssems[s], add=True)

    for t in range(NSLOT):
        stage(t, t, True)

    n_full = (N_BLK - NSLOT + 1) // NSLOT

    def triple(p, _):
        b0 = NSLOT * p
        for t in range(NSLOT):
            compute(t)
            bn = b0 + NSLOT + t

            @pl.when(bn < N_BLK)
            def _():
                stage(bn, t, False)
        return 0

    lax.fori_loop(0, n_full, triple, 0)
    for t in range(N_BLK - NSLOT * n_full):
        compute(t)
    for t in range(NSLOT):
        pltpu.make_async_copy(rows[t], agg_sh.at[dsts[t]], ssems[t]).wait()
    plsc.subcore_barrier()

    rbase = sid * ROWS_PER_TILE
    pltpu.sync_copy(agg_sh.at[pl.ds(rbase, ROWS_PER_TILE)],
                    out_hbm.at[cid].at[pl.ds(rbase, ROWS_PER_TILE)])


def _spmm(h, src, dst, adj):
    mesh = plsc.VectorSubcoreMesh(core_axis_name="c", subcore_axis_name="s")
    k = functools.partial(
        pl.kernel,
        out_type=jax.ShapeDtypeStruct((NC, AGG_ROWS, HID_DIM), jnp.float32),
        mesh=mesh,
        scratch_types=[
            pltpu.VMEM((E_BLK,), jnp.int32),
            pltpu.VMEM((E_BLK,), jnp.int32),
            pltpu.VMEM((E_BLK,), jnp.float32),
            pltpu.VMEM((E_BLK, HID_DIM), jnp.float32),
            pltpu.VMEM((E_BLK,), jnp.int32),
            pltpu.VMEM((E_BLK,), jnp.int32),
            pltpu.VMEM((E_BLK,), jnp.float32),
            pltpu.VMEM((E_BLK, HID_DIM), jnp.float32),
            pltpu.VMEM((E_BLK,), jnp.int32),
            pltpu.VMEM((E_BLK,), jnp.int32),
            pltpu.VMEM((E_BLK,), jnp.float32),
            pltpu.VMEM((E_BLK, HID_DIM), jnp.float32),
            pltpu.VMEM((ZCHUNK, HID_DIM), jnp.float32),
            pltpu.VMEM_SHARED((AGG_ROWS, HID_DIM), jnp.float32),
            pltpu.SemaphoreType.DMA,
            pltpu.SemaphoreType.DMA,
            pltpu.SemaphoreType.DMA,
            pltpu.SemaphoreType.DMA,
            pltpu.SemaphoreType.DMA,
            pltpu.SemaphoreType.DMA,
        ],
    )(_spmm_body)
    return k(h, src, dst, adj)


def _heads_body(p0_ref, p1_ref, wm1_ref, bm1_ref, wm2_ref, bm2_ref,
                ws1_ref, bs1_ref, ws2_ref, bs2_ref, mean_ref, std_ref):
    hidden = jnp.maximum(p0_ref[...] + p1_ref[...], 0.0)
    dn = (((1,), (1,)), ((), ()))
    a = jnp.maximum(
        lax.dot_general(hidden, wm1_ref[...], dn, preferred_element_type=jnp.float32)
        + bm1_ref[...][None, :], 0.0)
    mean_ref[...] = (lax.dot_general(a, wm2_ref[...], dn, preferred_element_type=jnp.float32)
                     + bm2_ref[...][None, :])
    s = jnp.maximum(
        lax.dot_general(hidden, ws1_ref[...], dn, preferred_element_type=jnp.float32)
        + bs1_ref[...][None, :], 0.0)
    pre = (lax.dot_general(s, ws2_ref[...], dn, preferred_element_type=jnp.float32)
           + bs2_ref[...][None, :])
    std_ref[...] = jnp.maximum(pre, 0.0) + jnp.log1p(jnp.exp(-jnp.abs(pre)))


def _heads(partials, wm1, bm1, wm2, bm2, ws1, bs1, ws2, bs2):
    blk = 1024
    grid = AGG_ROWS // blk
    wspec = lambda shape: pl.BlockSpec(shape, lambda i: tuple(0 for _ in shape))
    mean, std = pl.pallas_call(
        _heads_body,
        grid=(grid,),
        in_specs=[
            pl.BlockSpec((blk, HID_DIM), lambda i: (i, 0)),
            pl.BlockSpec((blk, HID_DIM), lambda i: (i, 0)),
            wspec((Z_DIM, HID_DIM)), wspec((Z_DIM,)),
            wspec((Z_DIM, Z_DIM)), wspec((Z_DIM,)),
            wspec((Z_DIM, HID_DIM)), wspec((Z_DIM,)),
            wspec((Z_DIM, Z_DIM)), wspec((Z_DIM,)),
        ],
        out_specs=[
            pl.BlockSpec((blk, Z_DIM), lambda i: (i, 0)),
            pl.BlockSpec((blk, Z_DIM), lambda i: (i, 0)),
        ],
        out_shape=[
            jax.ShapeDtypeStruct((AGG_ROWS, Z_DIM), jnp.float32),
            jax.ShapeDtypeStruct((AGG_ROWS, Z_DIM), jnp.float32),
        ],
    )(partials[0], partials[1],
      wm1, bm1, wm2, bm2, ws1, bs1, ws2, bs2)
    return mean, std


def kernel(x, edge_index, adj_values, W_shared, b_shared,
           W_m1, b_m1, W_m2, b_m2, W_s1, b_s1, W_s2, b_s2):
    ei = edge_index.astype(jnp.int32)
    dst = ei[0]
    src = ei[1]
    h = _shared_linear(x, W_shared, b_shared)
    partials = _spmm(h, src, dst, adj_values)
    mean, std = _heads(partials, W_m1, b_m1, W_m2, b_m2, W_s1, b_s1, W_s2, b_s2)
    mean = mean[:N_NODES]
    std = std[:N_NODES]
    return (mean, mean, std)
